# baseline (speedup 1.0000x reference)
import jax
import jax.numpy as jnp
from jax import lax
from jax.experimental import pallas as pl
from jax.experimental.pallas import tpu as pltpu
from jax.experimental.pallas import tpu_sc as plsc

_NUM_CORES = 2
_NUM_SUBCORES = 16
_NUM_WORKERS = _NUM_CORES * _NUM_SUBCORES
_CHUNK_ROWS = 32


def _sc_copy_body(pos_hbm, out_hbm, spbuf, si0, si1, so0, so1):
    rows_per_w = pos_hbm.shape[0] // _NUM_WORKERS
    n_chunks = rows_per_w // _CHUNK_ROWS
    wid = lax.axis_index("s") * _NUM_CORES + lax.axis_index("c")
    sid = lax.axis_index("s")
    base = wid * rows_per_w
    bufs = (spbuf.at[sid, 0], spbuf.at[sid, 1])
    sin, sout = (si0, si1), (so0, so1)

    def src(k):
        return pos_hbm.at[pl.ds(base + k * _CHUNK_ROWS, _CHUNK_ROWS)]

    def dst(k):
        return out_hbm.at[pl.ds(base + k * _CHUNK_ROWS, _CHUNK_ROWS)]

    cin = [None] * n_chunks
    cout = [None] * n_chunks
    cin[0] = pltpu.async_copy(src(0), bufs[0], sin[0])
    for k in range(n_chunks):
        b = k & 1
        if k + 1 < n_chunks:
            if k >= 1:
                cout[k - 1].wait()
            cin[k + 1] = pltpu.async_copy(src(k + 1), bufs[(k + 1) & 1], sin[(k + 1) & 1])
        cin[k].wait()
        cout[k] = pltpu.async_copy(bufs[b], dst(k), sout[b])
    cout[n_chunks - 1].wait()
    if n_chunks >= 2:
        cout[n_chunks - 2].wait()


def kernel(x, pos_table):
    seq_len = x.shape[1]
    d_model = pos_table.shape[1]
    mesh = plsc.VectorSubcoreMesh(core_axis_name="c", subcore_axis_name="s")
    k = pl.kernel(
        _sc_copy_body,
        out_type=jax.ShapeDtypeStruct((seq_len, d_model), pos_table.dtype),
        mesh=mesh,
        scratch_types=[
            pltpu.VMEM_SHARED(
                (_NUM_SUBCORES, 2, _CHUNK_ROWS, d_model), pos_table.dtype
            ),
            pltpu.SemaphoreType.DMA,
            pltpu.SemaphoreType.DMA,
            pltpu.SemaphoreType.DMA,
            pltpu.SemaphoreType.DMA,
        ],
    )
    return k(pos_table)


# speedup vs baseline: 2.0602x; 2.0602x over previous
import jax
import jax.numpy as jnp
from jax.experimental import pallas as pl
from jax.experimental.pallas import tpu as pltpu


def _embed_kernel(pos_ref, out_ref):
    out_ref[...] = pos_ref[...]


def kernel(x, pos_table):
    seq_len = x.shape[1]
    d_model = pos_table.shape[1]
    block_rows = 2048
    grid = seq_len // block_rows
    return pl.pallas_call(
        _embed_kernel,
        out_shape=jax.ShapeDtypeStruct((seq_len, d_model), pos_table.dtype),
        grid=(grid,),
        in_specs=[pl.BlockSpec((block_rows, d_model), lambda i: (i, 0))],
        out_specs=pl.BlockSpec((block_rows, d_model), lambda i: (i, 0)),
        compiler_params=pltpu.CompilerParams(
            dimension_semantics=("parallel",),
        ),
    )(pos_table)
